# Initial kernel scaffold; baseline (speedup 1.0000x reference)
import jax
import jax.numpy as jnp
from jax.experimental import pallas as pl


def kernel(x, token_table, pos_table):
    raise NotImplementedError("write your pallas kernel here")


# speedup vs baseline: 4.2509x; 4.2509x over previous
import functools

import jax
import jax.numpy as jnp
from jax import lax
from jax.experimental import pallas as pl
from jax.experimental.pallas import tpu as pltpu
from jax.experimental.pallas import tpu_sc as plsc

D = 128
B = 1024
S = 200
L = 16
NC, NS = 2, 16
NW = NC * NS
ROWS_PER_W = B // NW
GCHUNK = 40
NG = S // GCHUNK
FLAT = B * S


@jax.jit
def _sc_embed(x_flat, token_table, pos_table):
    mesh = plsc.VectorSubcoreMesh(core_axis_name="c", subcore_axis_name="s")

    @functools.partial(
        pl.kernel,
        mesh=mesh,
        out_type=jax.ShapeDtypeStruct((FLAT, D), jnp.float32),
        scratch_types=[
            pltpu.VMEM((S * ROWS_PER_W,), jnp.int32),
            pltpu.VMEM((S, D), jnp.float32),
            pltpu.VMEM((S, D), jnp.float32),
            pltpu.SemaphoreType.DMA,
        ],
    )
    def k(tok_hbm, pos_hbm, idx_hbm, out_hbm, idx_v, pos_v, buf, sem):
        wid = lax.axis_index("s") * NC + lax.axis_index("c")
        wbase = wid * (S * ROWS_PER_W)
        pltpu.sync_copy(idx_hbm.at[pl.ds(wbase, S * ROWS_PER_W)], idx_v)
        pltpu.sync_copy(pos_hbm, pos_v)

        @pl.loop(0, ROWS_PER_W)
        def _(r):
            copies = []
            for j in range(NG):
                copies.append(pltpu.async_copy(
                    tok_hbm.at[idx_v.at[pl.ds(r * S + j * GCHUNK, GCHUNK)]],
                    buf.at[pl.ds(j * GCHUNK, GCHUNK)],
                    sem,
                ))
            for c in copies:
                c.wait()

            @pl.loop(0, S)
            def _(i):
                for c in range(0, D, L):
                    buf[i, pl.ds(c, L)] += pos_v[i, pl.ds(c, L)]

            pltpu.sync_copy(buf, out_hbm.at[pl.ds(wbase + r * S, S)])

    return k(token_table, pos_table, x_flat)


def kernel(x, token_table, pos_table):
    x_flat = x.reshape(FLAT).astype(jnp.int32)
    out = _sc_embed(x_flat, token_table, pos_table)
    return out.reshape(B, S, D)
